# baseline (speedup 1.0000x reference)
import functools

import jax
import jax.numpy as jnp
from jax import lax
from jax.experimental import pallas as pl
from jax.experimental.pallas import tpu as pltpu
from jax.experimental.pallas import tpu_sc as plsc


@functools.lru_cache(maxsize=None)
def _build(total: int):
    info = plsc.get_sparse_core_info()
    nc, ns = 1, info.num_subcores
    nw = nc * ns
    nchunk = 4
    assert total % (nchunk * nw) == 0
    per = total // nw
    chunk = per // nchunk
    mesh = plsc.VectorSubcoreMesh(
        core_axis_name="c", subcore_axis_name="s", num_cores=1
    )

    @functools.partial(
        pl.kernel,
        mesh=mesh,
        out_type=jax.ShapeDtypeStruct((total,), jnp.float32),
        scratch_types=[
            [pltpu.VMEM((chunk,), jnp.float32) for _ in range(nchunk)],
            [pltpu.SemaphoreType.DMA for _ in range(nchunk)],
        ],
    )
    def body(w_hbm, out_hbm, vs, ss):
        wid = lax.axis_index("s") * nc + lax.axis_index("c")
        base = wid * per
        gets = [
            pltpu.async_copy(w_hbm.at[pl.ds(base + i * chunk, chunk)], vs[i], ss[i])
            for i in range(nchunk)
        ]
        puts = []
        for i in range(nchunk):
            gets[i].wait()
            puts.append(
                pltpu.async_copy(
                    vs[i], out_hbm.at[pl.ds(base + i * chunk, chunk)], ss[i]
                )
            )
        for p in puts:
            p.wait()

    return body


def kernel(input, weights):
    n = input.shape[0]
    d = weights.shape[1]
    flat = jnp.reshape(weights, (-1,))
    out = _build(n * d)(flat)
    return jnp.reshape(out, (n, d))


# speedup vs baseline: 1.0021x; 1.0021x over previous
import functools

import jax
import jax.numpy as jnp
from jax import lax
from jax.experimental import pallas as pl
from jax.experimental.pallas import tpu as pltpu
from jax.experimental.pallas import tpu_sc as plsc


@functools.lru_cache(maxsize=None)
def _build(total: int):
    info = plsc.get_sparse_core_info()
    nc, ns = 1, info.num_subcores
    nw = nc * ns
    nchunk = 1
    assert total % (nchunk * nw) == 0
    per = total // nw
    chunk = per // nchunk
    mesh = plsc.VectorSubcoreMesh(
        core_axis_name="c", subcore_axis_name="s", num_cores=1
    )

    @functools.partial(
        pl.kernel,
        mesh=mesh,
        out_type=jax.ShapeDtypeStruct((total,), jnp.float32),
        scratch_types=[
            [pltpu.VMEM((chunk,), jnp.float32) for _ in range(nchunk)],
            [pltpu.SemaphoreType.DMA for _ in range(nchunk)],
        ],
    )
    def body(w_hbm, out_hbm, vs, ss):
        wid = lax.axis_index("s") * nc + lax.axis_index("c")
        base = wid * per
        gets = [
            pltpu.async_copy(w_hbm.at[pl.ds(base + i * chunk, chunk)], vs[i], ss[i])
            for i in range(nchunk)
        ]
        puts = []
        for i in range(nchunk):
            gets[i].wait()
            puts.append(
                pltpu.async_copy(
                    vs[i], out_hbm.at[pl.ds(base + i * chunk, chunk)], ss[i]
                )
            )
        for p in puts:
            p.wait()

    return body


def kernel(input, weights):
    n = input.shape[0]
    d = weights.shape[1]
    flat = jnp.reshape(weights, (-1,))
    out = _build(n * d)(flat)
    return jnp.reshape(out, (n, d))


# speedup vs baseline: 1.0171x; 1.0150x over previous
import functools

import jax
import jax.numpy as jnp
from jax import lax
from jax.experimental import pallas as pl
from jax.experimental.pallas import tpu as pltpu
from jax.experimental.pallas import tpu_sc as plsc


@functools.lru_cache(maxsize=None)
def _build(total: int):
    info = plsc.get_sparse_core_info()
    nc, ns = 1, info.num_subcores
    nw = nc * ns
    nchunk = 2
    assert total % (nchunk * nw) == 0
    per = total // nw
    chunk = per // nchunk
    mesh = plsc.VectorSubcoreMesh(
        core_axis_name="c", subcore_axis_name="s", num_cores=1
    )

    @functools.partial(
        pl.kernel,
        mesh=mesh,
        out_type=jax.ShapeDtypeStruct((total,), jnp.float32),
        scratch_types=[
            [pltpu.VMEM((chunk,), jnp.float32) for _ in range(nchunk)],
            [pltpu.SemaphoreType.DMA for _ in range(nchunk)],
        ],
    )
    def body(w_hbm, out_hbm, vs, ss):
        wid = lax.axis_index("s") * nc + lax.axis_index("c")
        base = wid * per
        gets = [
            pltpu.async_copy(w_hbm.at[pl.ds(base + i * chunk, chunk)], vs[i], ss[i])
            for i in range(nchunk)
        ]
        puts = []
        for i in range(nchunk):
            gets[i].wait()
            puts.append(
                pltpu.async_copy(
                    vs[i], out_hbm.at[pl.ds(base + i * chunk, chunk)], ss[i]
                )
            )
        for p in puts:
            p.wait()

    return body


def kernel(input, weights):
    n = input.shape[0]
    d = weights.shape[1]
    flat = jnp.reshape(weights, (-1,))
    out = _build(n * d)(flat)
    return jnp.reshape(out, (n, d))
